# baseline (speedup 1.0000x reference)
import functools

import jax
import jax.numpy as jnp
from jax import lax
from jax.experimental import pallas as pl
from jax.experimental.pallas import tpu as pltpu
from jax.experimental.pallas import tpu_sc as plsc

_NUM_CORES = 2
_NUM_SUBCORES = 16
_NUM_WORKERS = _NUM_CORES * _NUM_SUBCORES

_PAD_N = 56
_SC_BATCHES = 2560
_TC_BLOCK = 8


@functools.partial(jax.jit, static_argnames=("batches_per_worker",))
def _sc_gather(embedding, padded_ids, batches_per_worker):
    num_padded = padded_ids.shape[0]
    d = embedding.shape[1]
    rows_per_worker = batches_per_worker * _PAD_N
    num_groups = batches_per_worker // 2
    mesh = plsc.VectorSubcoreMesh(
        core_axis_name="c",
        subcore_axis_name="s",
        num_cores=_NUM_CORES,
        num_subcores=_NUM_SUBCORES,
    )

    @functools.partial(
        pl.kernel,
        out_type=jax.ShapeDtypeStruct((num_padded, d), jnp.float32),
        mesh=mesh,
        scratch_types=[
            pltpu.VMEM((rows_per_worker,), jnp.int32),
            pltpu.VMEM((2, _PAD_N, d), jnp.float32),
            pltpu.SemaphoreType.DMA,
            pltpu.SemaphoreType.DMA,
        ],
    )
    def k(table_hbm, idx_hbm, out_hbm, idx_v, buf_v, gsem0, gsem1):
        gsems = (gsem0, gsem1)
        wid = lax.axis_index("s") * _NUM_CORES + lax.axis_index("c")
        base = wid * rows_per_worker
        pltpu.sync_copy(idx_hbm.at[pl.ds(base, rows_per_worker)], idx_v)

        def start_gather(c, b):
            idx_chunk = idx_v.at[pl.ds(c * _PAD_N, _PAD_N)]
            pltpu.async_copy(table_hbm.at[idx_chunk], buf_v.at[b], gsems[b])

        def wait_gather(c, b):
            idx_chunk = idx_v.at[pl.ds(c * _PAD_N, _PAD_N)]
            pltpu.make_async_copy(
                table_hbm.at[idx_chunk], buf_v.at[b], gsems[b]).wait()

        def scatter(c, b):
            off = pl.multiple_of(base + c * _PAD_N, 8)
            pltpu.sync_copy(buf_v.at[b], out_hbm.at[pl.ds(off, _PAD_N)])

        start_gather(0, 0)

        def body(g, _):
            c = 2 * g
            start_gather(c + 1, 1)
            wait_gather(c, 0)
            scatter(c, 0)
            start_gather(c + 2, 0)
            wait_gather(c + 1, 1)
            scatter(c + 1, 1)
            return _

        lax.fori_loop(0, num_groups - 1, body, None)

        c = batches_per_worker - 2
        start_gather(c + 1, 1)
        wait_gather(c, 0)
        scatter(c, 0)
        wait_gather(c + 1, 1)
        scatter(c + 1, 1)

    return k(embedding, padded_ids)


def _tc_gather(table_pad, ids_blocks):
    nb, flat = ids_blocks.shape
    v, d = table_pad.shape

    def body(ids_ref, tab_ref, o_ref):
        tab = tab_ref[...]
        for r in range(8):
            ids = ids_ref[r, :]
            onehot = (ids[:, None]
                      == lax.broadcasted_iota(jnp.int32, (flat, v), 1)
                      ).astype(jnp.float32)
            o_ref[pl.ds(r * flat, flat), :] = lax.dot_general(
                onehot, tab, (((1,), (0,)), ((), ())),
                precision=lax.Precision.HIGHEST,
                preferred_element_type=jnp.float32)

    return pl.pallas_call(
        body,
        grid=(nb // 8,),
        in_specs=[
            pl.BlockSpec((8, flat), lambda i: (i, 0)),
            pl.BlockSpec((v, d), lambda i: (0, 0)),
        ],
        out_specs=pl.BlockSpec((8 * flat, d), lambda i: (i, 0)),
        out_shape=jax.ShapeDtypeStruct((nb * flat, d), jnp.float32),
    )(ids_blocks, table_pad)


def kernel(relation_ids, embedding):
    b, n = relation_ids.shape
    d = embedding.shape[1]
    assert _SC_BATCHES % (2 * _NUM_WORKERS) == 0 and n <= _PAD_N
    assert (b - _SC_BATCHES) % _TC_BLOCK == 0
    table = embedding.astype(jnp.float32)
    ids = relation_ids.astype(jnp.int32)
    padded = jnp.pad(ids, ((0, 0), (0, _PAD_N - n)), mode="wrap")
    slab_sc = _sc_gather(table, padded[:_SC_BATCHES].reshape(-1),
                         _SC_BATCHES // _NUM_WORKERS)
    table_pad = jnp.pad(table, ((0, 128 - table.shape[0]), (0, 0)))
    tc_ids = padded[_SC_BATCHES:].reshape(-1, _TC_BLOCK * _PAD_N)
    slab_tc = _tc_gather(table_pad, tc_ids)
    slab = jnp.concatenate([slab_sc, slab_tc], axis=0)
    return slab.reshape(b, _PAD_N, d)[:, :n, :]


# speedup vs baseline: 1.1462x; 1.1462x over previous
import functools

import jax
import jax.numpy as jnp
from jax import lax
from jax.experimental import pallas as pl
from jax.experimental.pallas import tpu as pltpu
from jax.experimental.pallas import tpu_sc as plsc

_NUM_CORES = 2
_NUM_SUBCORES = 16
_NUM_WORKERS = _NUM_CORES * _NUM_SUBCORES

_PAD_N = 56


@functools.partial(jax.jit, static_argnames=("batches_per_worker", "n"))
def _sc_gather(embedding, padded_ids, batches_per_worker, n):
    num_padded = padded_ids.shape[0]
    d = embedding.shape[1]
    rows_per_worker = batches_per_worker * _PAD_N
    num_groups = batches_per_worker // 2
    mesh = plsc.VectorSubcoreMesh(
        core_axis_name="c",
        subcore_axis_name="s",
        num_cores=_NUM_CORES,
        num_subcores=_NUM_SUBCORES,
    )

    @functools.partial(
        pl.kernel,
        out_type=jax.ShapeDtypeStruct((num_padded, d), jnp.float32),
        mesh=mesh,
        scratch_types=[
            pltpu.VMEM((rows_per_worker,), jnp.int32),
            pltpu.VMEM((2, _PAD_N, d), jnp.float32),
            pltpu.SemaphoreType.DMA,
            pltpu.SemaphoreType.DMA,
            pltpu.SemaphoreType.DMA,
            pltpu.SemaphoreType.DMA,
        ],
    )
    def k(table_hbm, idx_hbm, out_hbm, idx_v, buf_v, gsem0, gsem1, ssem0, ssem1):
        gsems = (gsem0, gsem1)
        ssems = (ssem0, ssem1)
        wid = lax.axis_index("s") * _NUM_CORES + lax.axis_index("c")
        base = wid * rows_per_worker
        pltpu.sync_copy(idx_hbm.at[pl.ds(base, rows_per_worker)], idx_v)

        def start_gather(c, b):
            idx_chunk = idx_v.at[pl.ds(c * _PAD_N, _PAD_N)]
            pltpu.async_copy(table_hbm.at[idx_chunk], buf_v.at[b], gsems[b])

        def wait_gather(c, b):
            idx_chunk = idx_v.at[pl.ds(c * _PAD_N, _PAD_N)]
            pltpu.make_async_copy(
                table_hbm.at[idx_chunk], buf_v.at[b], gsems[b]).wait()

        def scatter_dma(c, b):
            off = pl.multiple_of(base + c * _PAD_N, 8)
            return pltpu.make_async_copy(
                buf_v.at[b], out_hbm.at[pl.ds(off, _PAD_N)], ssems[b])

        start_gather(0, 0)
        start_gather(1, 1)

        def body(g, _):
            c = 2 * g
            wait_gather(c, 0)
            scatter_dma(c, 0).start()
            wait_gather(c + 1, 1)
            scatter_dma(c + 1, 1).start()
            scatter_dma(c, 0).wait()
            start_gather(c + 2, 0)
            scatter_dma(c + 1, 1).wait()
            start_gather(c + 3, 1)
            return _

        lax.fori_loop(0, num_groups - 1, body, None)

        c = batches_per_worker - 2
        wait_gather(c, 0)
        scatter_dma(c, 0).start()
        wait_gather(c + 1, 1)
        scatter_dma(c + 1, 1).start()
        scatter_dma(c, 0).wait()
        scatter_dma(c + 1, 1).wait()

    return k(embedding, padded_ids)


def kernel(relation_ids, embedding):
    b, n = relation_ids.shape
    d = embedding.shape[1]
    assert b % _NUM_WORKERS == 0 and n <= _PAD_N
    ids = relation_ids.astype(jnp.int32)
    padded = jnp.pad(ids, ((0, 0), (0, _PAD_N - n)), mode="wrap").reshape(-1)
    out = _sc_gather(embedding.astype(jnp.float32), padded, b // _NUM_WORKERS, n)
    return out.reshape(b, _PAD_N, d)[:, :n, :]


# speedup vs baseline: 1.1491x; 1.0025x over previous
import functools

import jax
import jax.numpy as jnp
from jax import lax
from jax.experimental import pallas as pl
from jax.experimental.pallas import tpu as pltpu
from jax.experimental.pallas import tpu_sc as plsc

_NUM_CORES = 2
_NUM_SUBCORES = 16
_NUM_WORKERS = _NUM_CORES * _NUM_SUBCORES

_PAD_N = 56


@functools.partial(jax.jit, static_argnames=("batches_per_worker", "n"))
def _sc_gather(embedding, padded_ids, batches_per_worker, n):
    num_padded = padded_ids.shape[0]
    d = embedding.shape[1]
    rows_per_worker = batches_per_worker * _PAD_N
    num_groups = batches_per_worker // 2
    mesh = plsc.VectorSubcoreMesh(
        core_axis_name="c",
        subcore_axis_name="s",
        num_cores=_NUM_CORES,
        num_subcores=_NUM_SUBCORES,
    )

    @functools.partial(
        pl.kernel,
        out_type=jax.ShapeDtypeStruct((num_padded, d), jnp.float32),
        mesh=mesh,
        scratch_types=[
            pltpu.VMEM((rows_per_worker,), jnp.int32),
            pltpu.VMEM((2, _PAD_N, d), jnp.float32),
            pltpu.SemaphoreType.DMA,
            pltpu.SemaphoreType.DMA,
        ],
    )
    def k(table_hbm, idx_hbm, out_hbm, idx_v, buf_v, gsem0, gsem1):
        gsems = (gsem0, gsem1)
        wid = lax.axis_index("s") * _NUM_CORES + lax.axis_index("c")
        base = wid * rows_per_worker
        pltpu.sync_copy(idx_hbm.at[pl.ds(base, rows_per_worker)], idx_v)

        def start_gather(c, b):
            idx_chunk = idx_v.at[pl.ds(c * _PAD_N, _PAD_N)]
            pltpu.async_copy(table_hbm.at[idx_chunk], buf_v.at[b], gsems[b])

        def wait_gather(c, b):
            idx_chunk = idx_v.at[pl.ds(c * _PAD_N, _PAD_N)]
            pltpu.make_async_copy(
                table_hbm.at[idx_chunk], buf_v.at[b], gsems[b]).wait()

        def scatter(c, b):
            off = pl.multiple_of(base + c * _PAD_N, 8)
            pltpu.sync_copy(buf_v.at[b], out_hbm.at[pl.ds(off, _PAD_N)])

        start_gather(0, 0)

        def body(g, _):
            c = 2 * g
            start_gather(c + 1, 1)
            wait_gather(c, 0)
            scatter(c, 0)
            start_gather(c + 2, 0)
            wait_gather(c + 1, 1)
            scatter(c + 1, 1)
            return _

        lax.fori_loop(0, num_groups - 1, body, None)

        c = batches_per_worker - 2
        start_gather(c + 1, 1)
        wait_gather(c, 0)
        scatter(c, 0)
        wait_gather(c + 1, 1)
        scatter(c + 1, 1)

    return k(embedding, padded_ids)


def kernel(relation_ids, embedding):
    b, n = relation_ids.shape
    d = embedding.shape[1]
    assert b % _NUM_WORKERS == 0 and n <= _PAD_N
    ids = relation_ids.astype(jnp.int32)
    padded = jnp.pad(ids, ((0, 0), (0, _PAD_N - n)), mode="wrap").reshape(-1)
    out = _sc_gather(embedding.astype(jnp.float32), padded, b // _NUM_WORKERS, n)
    return out.reshape(b, _PAD_N, d)[:, :n, :]
